# Initial kernel scaffold; baseline (speedup 1.0000x reference)
import jax
import jax.numpy as jnp
from jax import lax
from jax.experimental import pallas as pl


_NEG = jnp.float32(-jnp.inf)
_BIG = jnp.int32(2**30)


def _topk3_body(a_ref, b_ref, o_ref):
    x = a_ref[...]
    ids = lax.broadcasted_iota(jnp.int32, x.shape, 1)

    m1 = jnp.max(x, axis=1, keepdims=True)
    i1 = jnp.min(jnp.where(x == m1, ids, _BIG), axis=1, keepdims=True)
    x = jnp.where(ids == i1, _NEG, x)

    m2 = jnp.max(x, axis=1, keepdims=True)
    i2 = jnp.min(jnp.where(x == m2, ids, _BIG), axis=1, keepdims=True)
    x = jnp.where(ids == i2, _NEG, x)

    m3 = jnp.max(x, axis=1, keepdims=True)

    c = lax.broadcasted_iota(jnp.int32, (x.shape[0], 128), 1)
    vals = jnp.where(c == 0, m1, jnp.where(c == 1, m2, jnp.where(c == 2, m3, 0.0)))
    o_ref[...] = vals * 2.0 + b_ref[...]


def kernel(a, b):
    m, n = a.shape
    b_row = jnp.zeros((1, 128), dtype=jnp.float32).at[0, :3].set(b)
    out = pl.pallas_call(
        _topk3_body,
        out_shape=jax.ShapeDtypeStruct((m, 128), jnp.float32),
    )(a, b_row)
    return out[:, :3]


# speedup vs baseline: 3.8214x; 3.8214x over previous
import jax
import jax.numpy as jnp
from jax import lax
from jax.experimental import pallas as pl


_NEG = float("-inf")
_BIG = 2**30


def _topk3_body(a_ref, b_ref, o_ref):
    x = a_ref[...]
    ids = lax.broadcasted_iota(jnp.int32, x.shape, 1)

    m1 = jnp.max(x, axis=1, keepdims=True)
    i1 = jnp.min(jnp.where(x == m1, ids, _BIG), axis=1, keepdims=True)
    x = jnp.where(ids == i1, _NEG, x)

    m2 = jnp.max(x, axis=1, keepdims=True)
    i2 = jnp.min(jnp.where(x == m2, ids, _BIG), axis=1, keepdims=True)
    x = jnp.where(ids == i2, _NEG, x)

    m3 = jnp.max(x, axis=1, keepdims=True)

    c = lax.broadcasted_iota(jnp.int32, (x.shape[0], 128), 1)
    vals = jnp.where(c == 0, m1, jnp.where(c == 1, m2, jnp.where(c == 2, m3, 0.0)))
    o_ref[...] = vals * 2.0 + b_ref[...]


def kernel(a, b):
    m, n = a.shape
    b_row = jnp.zeros((1, 128), dtype=jnp.float32).at[0, :3].set(b)
    out = pl.pallas_call(
        _topk3_body,
        out_shape=jax.ShapeDtypeStruct((m, 128), jnp.float32),
    )(a, b_row)
    return out[:, :3]
